# Initial kernel scaffold; baseline (speedup 1.0000x reference)
import functools

import jax
import jax.numpy as jnp
from jax import lax
from jax.experimental import pallas as pl
from jax.experimental.pallas import tpu as pltpu
from jax.experimental.pallas import tpu_sc as plsc

_NC = 2
_NS = 16
_NW = _NC * _NS
_L = 16

_DIM = 128
_VOCAB = 10


@functools.lru_cache(maxsize=None)
def _build_sc_gather(n_total: int):
    per_w = n_total // _NW
    n_chunks = 4
    chunk = per_w // n_chunks
    assert per_w % n_chunks == 0 and chunk % 8 == 0

    mesh = plsc.VectorSubcoreMesh(core_axis_name="c", subcore_axis_name="s")

    @functools.partial(
        pl.kernel,
        mesh=mesh,
        out_type=jax.ShapeDtypeStruct((n_total,), jnp.float32),
        scratch_types=[
            pltpu.VMEM((_VOCAB * _DIM,), jnp.float32),
            pltpu.VMEM((_DIM,), jnp.float32),
            pltpu.VMEM((_L,), jnp.float32),
            pltpu.VMEM((_L,), jnp.float32),
            pltpu.VMEM((chunk,), jnp.int32),
            pltpu.VMEM((chunk,), jnp.float32),
        ],
    )
    def sc_kernel(idx_hbm, tab_hbm, w_hbm, b_hbm, out_hbm,
                  tab_v, w_v, b_v, val_tab_v, idx_v, out_v):
        wid = lax.axis_index("s") * _NC + lax.axis_index("c")

        pltpu.sync_copy(tab_hbm, tab_v)
        pltpu.sync_copy(w_hbm, w_v)
        pltpu.sync_copy(b_hbm, b_v)

        lane = lax.iota(jnp.int32, _L)
        v_vec = jnp.zeros((_L,), jnp.float32)
        for k in range(_VOCAB):
            acc = tab_v[pl.ds(k * _DIM, _L)] * w_v[pl.ds(0, _L)]
            for c in range(1, _DIM // _L):
                acc = acc + (tab_v[pl.ds(k * _DIM + c * _L, _L)]
                             * w_v[pl.ds(c * _L, _L)])
            s = jnp.sum(acc)
            v_vec = jnp.where(lane == k, s, v_vec)
        v_vec = v_vec + b_v[...]
        val_tab_v[...] = v_vec

        for g in range(n_chunks):
            base = wid * per_w + g * chunk
            pltpu.sync_copy(idx_hbm.at[pl.ds(base, chunk)], idx_v)

            def body(i, carry):
                off = i * _L
                idx = idx_v[pl.ds(off, _L)]
                out_v[pl.ds(off, _L)] = plsc.load_gather(val_tab_v, [idx])
                return carry

            lax.fori_loop(0, chunk // _L, body, 0)
            pltpu.sync_copy(out_v, out_hbm.at[pl.ds(base, chunk)])

    return sc_kernel


def kernel(input, emb_table, lin_w, lin_b):
    bsz, seq = input.shape
    n_total = bsz * seq
    idx = input.reshape(n_total).astype(jnp.int32)
    tab = emb_table.astype(jnp.float32).reshape(_VOCAB * _DIM)
    w = lin_w.astype(jnp.float32).reshape(_DIM)
    b = jnp.broadcast_to(lin_b.astype(jnp.float32).reshape(1), (_L,))
    out = _build_sc_gather(n_total)(idx, tab, w, b)
    return out.reshape(bsz, seq, 1)


# speedup vs baseline: 86.9439x; 86.9439x over previous
import functools

import jax
import jax.numpy as jnp
from jax import lax
from jax.experimental import pallas as pl
from jax.experimental.pallas import tpu as pltpu
from jax.experimental.pallas import tpu_sc as plsc

_NC = 2
_NS = 16
_NW = _NC * _NS
_L = 16

_DIM = 128
_VOCAB = 10


@functools.lru_cache(maxsize=None)
def _build_sc_gather(n_total: int):
    per_w = n_total // _NW
    n_chunks = 4
    chunk = per_w // n_chunks
    assert per_w % n_chunks == 0 and chunk % 8 == 0

    mesh = plsc.VectorSubcoreMesh(core_axis_name="c", subcore_axis_name="s")

    @functools.partial(
        pl.kernel,
        mesh=mesh,
        out_type=jax.ShapeDtypeStruct((n_total,), jnp.float32),
        scratch_types=[
            pltpu.VMEM((_DIM * _L,), jnp.float32),
            pltpu.VMEM((_DIM,), jnp.float32),
            pltpu.VMEM((_L,), jnp.float32),
            pltpu.VMEM((_L,), jnp.float32),
            pltpu.VMEM((chunk,), jnp.int32),
            pltpu.VMEM((chunk,), jnp.float32),
        ],
        compiler_params=pltpu.CompilerParams(needs_layout_passes=False),
    )
    def sc_kernel(idx_hbm, tabt_hbm, w_hbm, b_hbm, out_hbm,
                  tabt_v, w_v, b_v, val_tab_v, idx_v, out_v):
        wid = lax.axis_index("s") * _NC + lax.axis_index("c")

        pltpu.sync_copy(tabt_hbm, tabt_v)
        pltpu.sync_copy(w_hbm, w_v)
        pltpu.sync_copy(b_hbm, b_v)

        v_vec = b_v[...]
        for dg in range(_DIM // _L):
            w_vec = w_v[pl.ds(dg * _L, _L)]
            for j in range(_L):
                d = dg * _L + j
                v_vec = v_vec + tabt_v[pl.ds(d * _L, _L)] * w_vec[j]
        val_tab_v[...] = v_vec

        for g in range(n_chunks):
            base = wid * per_w + g * chunk
            pltpu.sync_copy(idx_hbm.at[pl.ds(base, chunk)], idx_v)

            def body(i, carry):
                off = i * _L
                idx = idx_v[pl.ds(off, _L)]
                out_v[pl.ds(off, _L)] = plsc.load_gather(val_tab_v, [idx])
                return carry

            lax.fori_loop(0, chunk // _L, body, 0)
            pltpu.sync_copy(out_v, out_hbm.at[pl.ds(base, chunk)])

    return sc_kernel


def kernel(input, emb_table, lin_w, lin_b):
    bsz, seq = input.shape
    n_total = bsz * seq
    idx = input.reshape(n_total).astype(jnp.int32)
    tabt = jnp.zeros((_DIM, _L), jnp.float32)
    tabt = tabt.at[:, :_VOCAB].set(emb_table.astype(jnp.float32).T)
    tabt = tabt.reshape(_DIM * _L)
    w = lin_w.astype(jnp.float32).reshape(_DIM)
    b = jnp.broadcast_to(lin_b.astype(jnp.float32).reshape(1), (_L,))
    out = _build_sc_gather(n_total)(idx, tabt, w, b)
    return out.reshape(bsz, seq, 1)


# speedup vs baseline: 115.8038x; 1.3319x over previous
import functools

import jax
import jax.numpy as jnp
from jax import lax
from jax.experimental import pallas as pl
from jax.experimental.pallas import tpu as pltpu
from jax.experimental.pallas import tpu_sc as plsc

_NC = 2
_NS = 16
_NW = _NC * _NS
_L = 16

_DIM = 128
_VOCAB = 10


@functools.lru_cache(maxsize=None)
def _build_sc_gather(n_total: int):
    per_w = n_total // _NW
    n_chunks = 4
    chunk = per_w // n_chunks
    assert per_w % n_chunks == 0 and chunk % 8 == 0

    mesh = plsc.VectorSubcoreMesh(core_axis_name="c", subcore_axis_name="s")

    @functools.partial(
        pl.kernel,
        mesh=mesh,
        out_type=jax.ShapeDtypeStruct((n_total,), jnp.float32),
        scratch_types=[
            pltpu.VMEM((_DIM * _L,), jnp.float32),
            pltpu.VMEM((_DIM,), jnp.float32),
            pltpu.VMEM((_L,), jnp.float32),
            pltpu.VMEM((_L,), jnp.float32),
            pltpu.VMEM((chunk,), jnp.int32),
            pltpu.VMEM((chunk,), jnp.int32),
            pltpu.VMEM((chunk,), jnp.float32),
            pltpu.VMEM((chunk,), jnp.float32),
            pltpu.SemaphoreType.DMA,
            pltpu.SemaphoreType.DMA,
            pltpu.SemaphoreType.DMA,
            pltpu.SemaphoreType.DMA,
        ],
        compiler_params=pltpu.CompilerParams(needs_layout_passes=False),
    )
    def sc_kernel(idx_hbm, tabt_hbm, w_hbm, b_hbm, out_hbm,
                  tabt_v, w_v, b_v, val_tab_v,
                  idx_v0, idx_v1, out_v0, out_v1,
                  in_sem0, in_sem1, out_sem0, out_sem1):
        wid = lax.axis_index("s") * _NC + lax.axis_index("c")

        pltpu.sync_copy(tabt_hbm, tabt_v)
        pltpu.sync_copy(w_hbm, w_v)
        pltpu.sync_copy(b_hbm, b_v)

        v_vec = b_v[...]
        for dg in range(_DIM // _L):
            w_vec = w_v[pl.ds(dg * _L, _L)]
            for j in range(_L):
                d = dg * _L + j
                v_vec = v_vec + tabt_v[pl.ds(d * _L, _L)] * w_vec[j]
        val_tab_v[...] = v_vec

        idx_bufs = [idx_v0, idx_v1]
        out_bufs = [out_v0, out_v1]
        in_sems = [in_sem0, in_sem1]
        out_sems = [out_sem0, out_sem1]
        base0 = wid * per_w

        in_copies = [None] * n_chunks
        out_copies = [None] * n_chunks
        in_copies[0] = pltpu.async_copy(
            idx_hbm.at[pl.ds(base0, chunk)], idx_bufs[0], in_sems[0])
        for g in range(n_chunks):
            b = g % 2
            if g + 1 < n_chunks:
                nb = (g + 1) % 2
                in_copies[g + 1] = pltpu.async_copy(
                    idx_hbm.at[pl.ds(base0 + (g + 1) * chunk, chunk)],
                    idx_bufs[nb], in_sems[nb])
            in_copies[g].wait()
            if g >= 2:
                out_copies[g - 2].wait()

            idx_b = idx_bufs[b]
            out_b = out_bufs[b]

            @plsc.parallel_loop(0, chunk, step=_L, unroll=8)
            def body(i, idx_b=idx_b, out_b=out_b):
                idx = idx_b[pl.ds(i, _L)]
                out_b[pl.ds(i, _L)] = plsc.load_gather(val_tab_v, [idx])

            out_copies[g] = pltpu.async_copy(
                out_bufs[b], out_hbm.at[pl.ds(base0 + g * chunk, chunk)],
                out_sems[b])
        out_copies[n_chunks - 2].wait()
        out_copies[n_chunks - 1].wait()

    return sc_kernel


def kernel(input, emb_table, lin_w, lin_b):
    bsz, seq = input.shape
    n_total = bsz * seq
    idx = input.reshape(n_total).astype(jnp.int32)
    tabt = jnp.zeros((_DIM, _L), jnp.float32)
    tabt = tabt.at[:, :_VOCAB].set(emb_table.astype(jnp.float32).T)
    tabt = tabt.reshape(_DIM * _L)
    w = lin_w.astype(jnp.float32).reshape(_DIM)
    b = jnp.broadcast_to(lin_b.astype(jnp.float32).reshape(1), (_L,))
    out = _build_sc_gather(n_total)(idx, tabt, w, b)
    return out.reshape(bsz, seq, 1)


# speedup vs baseline: 187.7309x; 1.6211x over previous
import functools

import jax
import jax.numpy as jnp
from jax import lax
from jax.experimental import pallas as pl
from jax.experimental.pallas import tpu as pltpu
from jax.experimental.pallas import tpu_sc as plsc

_NC = 2
_NS = 16
_NW = _NC * _NS
_L = 16

_DIM = 128
_VOCAB = 10


@functools.lru_cache(maxsize=None)
def _build_sc_gather(n_rows: int, n_cols: int):
    rows_per_w = n_rows // _NW
    n_chunks = 8
    crows = rows_per_w // n_chunks
    assert rows_per_w % n_chunks == 0 and crows % 8 == 0

    col_offs = list(range(0, n_cols - _L + 1, _L))
    if col_offs[-1] != n_cols - _L:
        col_offs.append(n_cols - _L)

    mesh = plsc.VectorSubcoreMesh(core_axis_name="c", subcore_axis_name="s")

    @functools.partial(
        pl.kernel,
        mesh=mesh,
        out_type=jax.ShapeDtypeStruct((n_rows, n_cols), jnp.float32),
        scratch_types=[
            pltpu.VMEM((_DIM * _L,), jnp.float32),
            pltpu.VMEM((_DIM,), jnp.float32),
            pltpu.VMEM((_L,), jnp.float32),
            pltpu.VMEM((_L,), jnp.float32),
            pltpu.VMEM((crows, n_cols), jnp.int32),
            pltpu.VMEM((crows, n_cols), jnp.int32),
            pltpu.VMEM((crows, n_cols), jnp.float32),
            pltpu.VMEM((crows, n_cols), jnp.float32),
            pltpu.SemaphoreType.DMA,
            pltpu.SemaphoreType.DMA,
            pltpu.SemaphoreType.DMA,
            pltpu.SemaphoreType.DMA,
        ],
        compiler_params=pltpu.CompilerParams(
            needs_layout_passes=False, use_tc_tiling_on_sc=True),
    )
    def sc_kernel(idx_hbm, tabt_hbm, w_hbm, b_hbm, out_hbm,
                  tabt_v, w_v, b_v, val_tab_v,
                  idx_v0, idx_v1, out_v0, out_v1,
                  in_sem0, in_sem1, out_sem0, out_sem1):
        wid = lax.axis_index("s") * _NC + lax.axis_index("c")

        pltpu.sync_copy(tabt_hbm, tabt_v)
        pltpu.sync_copy(w_hbm, w_v)
        pltpu.sync_copy(b_hbm, b_v)

        v_vec = b_v[...]
        for dg in range(_DIM // _L):
            w_vec = w_v[pl.ds(dg * _L, _L)]
            for j in range(_L):
                d = dg * _L + j
                v_vec = v_vec + tabt_v[pl.ds(d * _L, _L)] * w_vec[j]
        val_tab_v[...] = v_vec

        idx_bufs = [idx_v0, idx_v1]
        out_bufs = [out_v0, out_v1]
        in_sems = [in_sem0, in_sem1]
        out_sems = [out_sem0, out_sem1]
        row0 = wid * rows_per_w

        in_copies = [None] * n_chunks
        out_copies = [None] * n_chunks
        in_copies[0] = pltpu.async_copy(
            idx_hbm.at[pl.ds(row0, crows), :], idx_bufs[0], in_sems[0])
        for g in range(n_chunks):
            b = g % 2
            if g + 1 < n_chunks:
                nb = (g + 1) % 2
                in_copies[g + 1] = pltpu.async_copy(
                    idx_hbm.at[pl.ds(row0 + (g + 1) * crows, crows), :],
                    idx_bufs[nb], in_sems[nb])
            in_copies[g].wait()
            if g >= 2:
                out_copies[g - 2].wait()

            idx_b = idx_bufs[b]
            out_b = out_bufs[b]

            @plsc.parallel_loop(0, crows, step=1, unroll=2)
            def body(r, idx_b=idx_b, out_b=out_b):
                for off in col_offs:
                    idx = idx_b[r, pl.ds(off, _L)]
                    out_b[r, pl.ds(off, _L)] = plsc.load_gather(
                        val_tab_v, [idx])

            out_copies[g] = pltpu.async_copy(
                out_bufs[b], out_hbm.at[pl.ds(row0 + g * crows, crows), :],
                out_sems[b])
        out_copies[n_chunks - 2].wait()
        out_copies[n_chunks - 1].wait()

    return sc_kernel


def kernel(input, emb_table, lin_w, lin_b):
    bsz, seq = input.shape
    idx = input.astype(jnp.int32)
    tabt = jnp.zeros((_DIM, _L), jnp.float32)
    tabt = tabt.at[:, :_VOCAB].set(emb_table.astype(jnp.float32).T)
    tabt = tabt.reshape(_DIM * _L)
    w = lin_w.astype(jnp.float32).reshape(_DIM)
    b = jnp.broadcast_to(lin_b.astype(jnp.float32).reshape(1), (_L,))
    out = _build_sc_gather(bsz, seq)(idx, tabt, w, b)
    return out.reshape(bsz, seq, 1)


# speedup vs baseline: 205.5984x; 1.0952x over previous
import functools

import jax
import jax.numpy as jnp
from jax import lax
from jax.experimental import pallas as pl
from jax.experimental.pallas import tpu as pltpu
from jax.experimental.pallas import tpu_sc as plsc

_NC = 2
_NS = 16
_NW = _NC * _NS
_L = 16

_DIM = 128
_VOCAB = 10


@functools.lru_cache(maxsize=None)
def _build_sc_gather(n_rows: int, n_cols: int):
    assert n_rows % (_NW * 512) == 0 and n_cols % 8 == 0
    rwin = n_rows // _NW
    n_trg = n_cols // 8
    group = 5
    assert n_trg % group == 0
    n_units = n_trg // group
    grows = group * 8
    rt_per_w = rwin // 128

    mesh = plsc.VectorSubcoreMesh(core_axis_name="c", subcore_axis_name="s")

    @functools.partial(
        pl.kernel,
        mesh=mesh,
        out_type=jax.ShapeDtypeStruct((n_cols, n_rows // 128, 128),
                                      jnp.float32),
        scratch_types=[
            pltpu.VMEM((_DIM * _L,), jnp.float32),
            pltpu.VMEM((_DIM,), jnp.float32),
            pltpu.VMEM((_L,), jnp.float32),
            pltpu.VMEM((_L,), jnp.float32),
            pltpu.VMEM((grows, rwin), jnp.int32),
            pltpu.VMEM((grows, rwin), jnp.int32),
            pltpu.VMEM((grows, rt_per_w, 128), jnp.float32),
            pltpu.VMEM((grows, rt_per_w, 128), jnp.float32),
            pltpu.SemaphoreType.DMA,
            pltpu.SemaphoreType.DMA,
            pltpu.SemaphoreType.DMA,
            pltpu.SemaphoreType.DMA,
        ],
        compiler_params=pltpu.CompilerParams(
            needs_layout_passes=False, use_tc_tiling_on_sc=True),
    )
    def sc_kernel(idx_hbm, tabt_hbm, w_hbm, b_hbm, out_hbm,
                  tabt_v, w_v, b_v, val_tab_v,
                  idx_v0, idx_v1, out_v0, out_v1,
                  in_sem0, in_sem1, out_sem0, out_sem1):
        wid = lax.axis_index("s") * _NC + lax.axis_index("c")
        r0 = wid * rwin

        pltpu.sync_copy(tabt_hbm, tabt_v)
        pltpu.sync_copy(w_hbm, w_v)
        pltpu.sync_copy(b_hbm, b_v)

        v_vec = b_v[...]
        for dg in range(_DIM // _L):
            w_vec = w_v[pl.ds(dg * _L, _L)]
            for j in range(_L):
                d = dg * _L + j
                v_vec = v_vec + tabt_v[pl.ds(d * _L, _L)] * w_vec[j]
        val_tab_v[...] = v_vec

        idx_bufs = [idx_v0, idx_v1]
        out_bufs = [out_v0, out_v1]
        in_sems = [in_sem0, in_sem1]
        out_sems = [out_sem0, out_sem1]

        def start_in(u, b):
            return pltpu.async_copy(
                idx_hbm.at[pl.ds(u * grows, grows), pl.ds(r0, rwin)],
                idx_bufs[b], in_sems[b])

        in_copies = [None] * n_units
        out_copies = [None] * n_units
        in_copies[0] = start_in(0, 0)
        for u in range(n_units):
            b = u % 2
            if u + 1 < n_units:
                in_copies[u + 1] = start_in(u + 1, (u + 1) % 2)
            in_copies[u].wait()
            if u >= 2:
                out_copies[u - 2].wait()

            idx_b = idx_bufs[b]
            out_b = out_bufs[b]

            @plsc.parallel_loop(0, grows * rt_per_w, step=1, unroll=2)
            def body(q, idx_b=idx_b, out_b=out_b):
                row = q // rt_per_w
                t = q % rt_per_w
                for j2 in range(128 // _L):
                    idx = idx_b[row, pl.ds(t * 128 + j2 * _L, _L)]
                    out_b[row, t, pl.ds(j2 * _L, _L)] = plsc.load_gather(
                        val_tab_v, [idx])

            out_copies[u] = pltpu.async_copy(
                out_b,
                out_hbm.at[pl.ds(u * grows, grows),
                           pl.ds(wid * rt_per_w, rt_per_w), :],
                out_sems[b])
        out_copies[n_units - 2].wait()
        out_copies[n_units - 1].wait()

    return sc_kernel


def kernel(input, emb_table, lin_w, lin_b):
    bsz, seq = input.shape
    idx_t = input.astype(jnp.int32).T
    tabt = jnp.zeros((_DIM, _L), jnp.float32)
    tabt = tabt.at[:, :_VOCAB].set(emb_table.astype(jnp.float32).T)
    tabt = tabt.reshape(_DIM * _L)
    w = lin_w.astype(jnp.float32).reshape(_DIM)
    b = jnp.broadcast_to(lin_b.astype(jnp.float32).reshape(1), (_L,))
    cube = _build_sc_gather(bsz, seq)(idx_t, tabt, w, b)
    return cube.reshape(seq, bsz).T.reshape(bsz, seq, 1)


# speedup vs baseline: 309.4448x; 1.5051x over previous
import functools

import jax
import jax.numpy as jnp
from jax import lax
from jax.experimental import pallas as pl
from jax.experimental.pallas import tpu as pltpu
from jax.experimental.pallas import tpu_sc as plsc

_NC = 2
_NS = 16
_NW = _NC * _NS
_L = 16

_DIM = 128
_VOCAB = 10

_UNIT = 4096


@functools.lru_cache(maxsize=None)
def _build_sc_gather(n_rows: int, n_cols: int):
    q_per_line = n_rows // _UNIT
    n_units = n_cols * q_per_line
    assert n_rows % _UNIT == 0 and n_units % _NW == 0
    u_per_w = n_units // _NW

    mesh = plsc.VectorSubcoreMesh(core_axis_name="c", subcore_axis_name="s")

    @functools.partial(
        pl.kernel,
        mesh=mesh,
        out_type=jax.ShapeDtypeStruct((n_cols, 1, n_rows), jnp.float32),
        scratch_types=[
            pltpu.VMEM((_DIM * _L,), jnp.float32),
            pltpu.VMEM((_DIM,), jnp.float32),
            pltpu.VMEM((_L,), jnp.float32),
            pltpu.VMEM((_L,), jnp.float32),
            pltpu.VMEM((_UNIT,), jnp.int32),
            pltpu.VMEM((_UNIT,), jnp.int32),
            pltpu.VMEM((_UNIT,), jnp.float32),
            pltpu.VMEM((_UNIT,), jnp.float32),
            pltpu.SemaphoreType.DMA,
            pltpu.SemaphoreType.DMA,
            pltpu.SemaphoreType.DMA,
            pltpu.SemaphoreType.DMA,
        ],
        compiler_params=pltpu.CompilerParams(
            needs_layout_passes=False, use_tc_tiling_on_sc=True),
    )
    def sc_kernel(idx_hbm, tabt_hbm, w_hbm, b_hbm, out_hbm,
                  tabt_v, w_v, b_v, val_tab_v,
                  idx_v0, idx_v1, out_v0, out_v1,
                  in_sem0, in_sem1, out_sem0, out_sem1):
        wid = lax.axis_index("s") * _NC + lax.axis_index("c")
        u0 = wid * u_per_w

        pltpu.sync_copy(tabt_hbm, tabt_v)
        pltpu.sync_copy(w_hbm, w_v)
        pltpu.sync_copy(b_hbm, b_v)

        v_vec = b_v[...]
        for dg in range(_DIM // _L):
            w_vec = w_v[pl.ds(dg * _L, _L)]
            for j in range(_L):
                d = dg * _L + j
                v_vec = v_vec + tabt_v[pl.ds(d * _L, _L)] * w_vec[j]
        val_tab_v[...] = v_vec

        idx_bufs = [idx_v0, idx_v1]
        out_bufs = [out_v0, out_v1]
        in_sems = [in_sem0, in_sem1]
        out_sems = [out_sem0, out_sem1]

        def start_in(i, b):
            u = u0 + i
            line = u // q_per_line
            r_off = (u % q_per_line) * _UNIT
            return pltpu.async_copy(
                idx_hbm.at[line, pl.ds(r_off, _UNIT)], idx_bufs[b],
                in_sems[b])

        in_copies = [None] * u_per_w
        out_copies = [None] * u_per_w
        in_copies[0] = start_in(0, 0)
        for i in range(u_per_w):
            b = i % 2
            if i + 1 < u_per_w:
                in_copies[i + 1] = start_in(i + 1, (i + 1) % 2)
            in_copies[i].wait()
            if i >= 2:
                out_copies[i - 2].wait()

            idx_b = idx_bufs[b]
            out_b = out_bufs[b]

            @plsc.parallel_loop(0, _UNIT, step=_L, unroll=8)
            def body(j, idx_b=idx_b, out_b=out_b):
                out_b[pl.ds(j, _L)] = plsc.load_gather(
                    val_tab_v, [idx_b[pl.ds(j, _L)]])

            u = u0 + i
            out_copies[i] = pltpu.async_copy(
                out_b,
                out_hbm.at[u // q_per_line, 0,
                           pl.ds((u % q_per_line) * _UNIT, _UNIT)],
                out_sems[b])
        out_copies[u_per_w - 2].wait()
        out_copies[u_per_w - 1].wait()

    return sc_kernel


def kernel(input, emb_table, lin_w, lin_b):
    bsz, seq = input.shape
    idx_t = input.astype(jnp.int32).T
    tabt = jnp.zeros((_DIM, _L), jnp.float32)
    tabt = tabt.at[:, :_VOCAB].set(emb_table.astype(jnp.float32).T)
    tabt = tabt.reshape(_DIM * _L)
    w = lin_w.astype(jnp.float32).reshape(_DIM)
    b = jnp.broadcast_to(lin_b.astype(jnp.float32).reshape(1), (_L,))
    cube = _build_sc_gather(bsz, seq)(idx_t, tabt, w, b)
    return cube.transpose(2, 0, 1)
